# baseline (speedup 1.0000x reference)
import functools

import jax
import jax.numpy as jnp
from jax import lax
from jax.experimental import pallas as pl
from jax.experimental.pallas import tpu as pltpu
from jax.experimental.pallas import tpu_sc as plsc

_ID_NUM = 100000
_ID_DIM = 32
_EXP_DIM = 16
_BATCH = 16384

_NC, _NS, _L = 2, 16, 16
_NW = _NC * _NS
_B_PER_W = _BATCH // _NW
_CHUNK = 128
_NCHUNK = _B_PER_W // _CHUNK

_EXP_TILE_ROW = (800000 // 128) * 1024
_ID_PAD = 100096
_ID_TILE_ROW = (_ID_PAD // 128) * 1024
_EXP_TR = _EXP_DIM // 8
_ID_TR = _ID_DIM // 8
_OUT_TILE_ROW = (_BATCH // 128) * 1024

_EXP_ROWS = _NCHUNK * _EXP_TR * 8
_ID_ROWS = _NCHUNK * _ID_TR * 8
_EB_WORDS = _EXP_ROWS * _CHUNK
_IB_WORDS = _ID_ROWS * _CHUNK


@functools.cache
def _make_sc_call():
  mesh = plsc.VectorSubcoreMesh(core_axis_name="c", subcore_axis_name="s")

  @functools.partial(
      pl.kernel,
      mesh=mesh,
      out_type=(
          jax.ShapeDtypeStruct((_EXP_TR * _OUT_TILE_ROW,), jnp.float32),
          jax.ShapeDtypeStruct((_ID_TR * _OUT_TILE_ROW,), jnp.float32),
      ),
      scratch_types=[
          pltpu.VMEM((_NCHUNK, _CHUNK), jnp.int32),
          pltpu.VMEM((_NCHUNK, _CHUNK), jnp.int32),
          pltpu.VMEM((_B_PER_W,), jnp.int32),
          pltpu.VMEM((_B_PER_W,), jnp.int32),
          pltpu.VMEM((_EB_WORDS,), jnp.int32),
          pltpu.VMEM((_IB_WORDS,), jnp.int32),
          pltpu.VMEM((_EB_WORDS,), jnp.float32),
          pltpu.VMEM((_IB_WORDS,), jnp.float32),
          pltpu.SemaphoreType.DMA,
          pltpu.SemaphoreType.DMA,
      ],
  )
  def sc_kernel(exp_hbm, id_hbm, exp_flat_hbm, id_flat_hbm,
                exp_out_hbm, id_out_hbm,
                eidx_v, iidx_v, ebase_v, ibase_v, eoff_v, ioff_v,
                ebuf_v, ibuf_v, sem_e, sem_i):
    wid = lax.axis_index("s") * _NC + lax.axis_index("c")

    pltpu.sync_copy(exp_hbm.at[wid], eidx_v)
    pltpu.sync_copy(id_hbm.at[wid], iidx_v)

    for q in range(_NCHUNK):
      for g in range(_CHUNK // _L):
        sl = pl.ds(g * _L, _L)
        fsl = pl.ds(q * _CHUNK + g * _L, _L)
        i_v = iidx_v[q, sl]
        f_v = eidx_v[q, sl] * _ID_NUM + i_v
        ebase_v[fsl] = f_v + (f_v >> 7) * 896
        ibase_v[fsl] = i_v + (i_v >> 7) * 896

    def eoff_body(d):
      tr = d // (_NCHUNK * 8)
      q = (d // 8) % _NCHUNK
      r = d % 8
      c0 = tr * _EXP_TILE_ROW + r * 128
      for g in range(_CHUNK // _L):
        eoff_v[pl.ds(d * _CHUNK + g * _L, _L)] = (
            ebase_v[pl.ds(q * _CHUNK + g * _L, _L)] + c0)

    pl.loop(0, _EXP_ROWS)(eoff_body)

    def ioff_body(d):
      tr = d // (_NCHUNK * 8)
      q = (d // 8) % _NCHUNK
      r = d % 8
      c0 = tr * _ID_TILE_ROW + r * 128
      for g in range(_CHUNK // _L):
        ioff_v[pl.ds(d * _CHUNK + g * _L, _L)] = (
            ibase_v[pl.ds(q * _CHUNK + g * _L, _L)] + c0)

    pl.loop(0, _ID_ROWS)(ioff_body)

    trw = _NCHUNK * _CHUNK * 8
    for tr in range(_EXP_TR):
      pltpu.async_copy(
          exp_flat_hbm.at[eoff_v.at[pl.ds(tr * trw, trw)]],
          ebuf_v.at[pl.ds(tr * trw, trw)], sem_e)
    for tr in range(_ID_TR):
      pltpu.async_copy(
          id_flat_hbm.at[ioff_v.at[pl.ds(tr * trw, trw)]],
          ibuf_v.at[pl.ds(tr * trw, trw)], sem_i)

    pltpu.make_async_copy(
        exp_out_hbm.at[pl.ds(0, _EB_WORDS)], ebuf_v, sem_e).wait()
    pltpu.make_async_copy(
        id_out_hbm.at[pl.ds(0, _IB_WORDS)], ibuf_v, sem_i).wait()

    run = _NCHUNK * _CHUNK * 8
    for tr in range(_EXP_TR):
      pltpu.sync_copy(
          ebuf_v.at[pl.ds(tr * run, run)],
          exp_out_hbm.at[pl.ds(tr * _OUT_TILE_ROW + wid * run, run)])
    for tr in range(_ID_TR):
      pltpu.sync_copy(
          ibuf_v.at[pl.ds(tr * run, run)],
          id_out_hbm.at[pl.ds(tr * _OUT_TILE_ROW + wid * run, run)])

  return sc_kernel


def kernel(exp_infor, id_infor, id_table, exp_table):
  e3 = exp_infor.astype(jnp.int32).reshape(_NW, _NCHUNK, _CHUNK)
  i3 = id_infor.astype(jnp.int32).reshape(_NW, _NCHUNK, _CHUNK)

  exp_flat = (exp_table.reshape(6250, 128, _EXP_TR, 8)
              .transpose(2, 0, 3, 1).reshape(-1))
  id_pad = jnp.pad(id_table, ((0, _ID_PAD - _ID_NUM), (0, 0)))
  id_flat = (id_pad.reshape(_ID_PAD // 128, 128, _ID_TR, 8)
             .transpose(2, 0, 3, 1).reshape(-1))

  eo_flat, io_flat = _make_sc_call()(e3, i3, exp_flat, id_flat)

  exp_code = (eo_flat.reshape(_EXP_TR, 128, 8, 128)
              .transpose(1, 3, 0, 2).reshape(_BATCH, _EXP_DIM))
  id_code = (io_flat.reshape(_ID_TR, 128, 8, 128)
             .transpose(1, 3, 0, 2).reshape(_BATCH, _ID_DIM))
  return (exp_code, id_code)


# speedup vs baseline: 1.0787x; 1.0787x over previous
import functools

import jax
import jax.numpy as jnp
from jax import lax
from jax.experimental import pallas as pl
from jax.experimental.pallas import tpu as pltpu
from jax.experimental.pallas import tpu_sc as plsc

_ID_NUM = 100000
_ID_DIM = 32
_EXP_DIM = 16
_BATCH = 16384

_NC, _NS, _L = 2, 16, 16
_NW = _NC * _NS
_B_PER_W = _BATCH // _NW
_CHUNK = 128
_NCHUNK = _B_PER_W // _CHUNK

_EXP_TILE_ROW = (800000 // 128) * 1024
_ID_PAD = 100096
_ID_TILE_ROW = (_ID_PAD // 128) * 1024
_EXP_TR = _EXP_DIM // 8
_ID_TR = _ID_DIM // 8
_OUT_TILE_ROW = (_BATCH // 128) * 1024

_EXP_ROWS = _NCHUNK * _EXP_TR * 8
_ID_ROWS = _NCHUNK * _ID_TR * 8
_EB_WORDS = _EXP_ROWS * _CHUNK
_IB_WORDS = _ID_ROWS * _CHUNK


@functools.cache
def _make_sc_call():
  mesh = plsc.VectorSubcoreMesh(core_axis_name="c", subcore_axis_name="s")

  @functools.partial(
      pl.kernel,
      mesh=mesh,
      out_type=(
          jax.ShapeDtypeStruct((_EXP_TR * _OUT_TILE_ROW,), jnp.float32),
          jax.ShapeDtypeStruct((_ID_TR * _OUT_TILE_ROW,), jnp.float32),
      ),
      scratch_types=[
          pltpu.VMEM((_NCHUNK, _CHUNK), jnp.int32),
          pltpu.VMEM((_NCHUNK, _CHUNK), jnp.int32),
          pltpu.VMEM((_B_PER_W,), jnp.int32),
          pltpu.VMEM((_B_PER_W,), jnp.int32),
          pltpu.VMEM((_EB_WORDS,), jnp.int32),
          pltpu.VMEM((_IB_WORDS,), jnp.int32),
          pltpu.VMEM((_EB_WORDS,), jnp.float32),
          pltpu.VMEM((_IB_WORDS,), jnp.float32),
          pltpu.SemaphoreType.DMA,
          pltpu.SemaphoreType.DMA,
      ],
  )
  def sc_kernel(exp_hbm, id_hbm, exp_flat_hbm, id_flat_hbm,
                exp_out_hbm, id_out_hbm,
                eidx_v, iidx_v, ebase_v, ibase_v, eoff_v, ioff_v,
                ebuf_v, ibuf_v, sem_e, sem_i):
    wid = lax.axis_index("s") * _NC + lax.axis_index("c")

    pltpu.sync_copy(exp_hbm.at[wid], eidx_v)
    pltpu.sync_copy(id_hbm.at[wid], iidx_v)

    for q in range(_NCHUNK):
      for g in range(_CHUNK // _L):
        sl = pl.ds(g * _L, _L)
        fsl = pl.ds(q * _CHUNK + g * _L, _L)
        i_v = iidx_v[q, sl]
        ibase_v[fsl] = i_v + (i_v >> 7) * 896

    trw = _NCHUNK * _CHUNK * 8

    def ioff_body(d):
      tr = d // (_NCHUNK * 8)
      q = (d // 8) % _NCHUNK
      r = d % 8
      c0 = tr * _ID_TILE_ROW + r * 128
      for g in range(_CHUNK // _L):
        ioff_v[pl.ds(d * _CHUNK + g * _L, _L)] = (
            ibase_v[pl.ds(q * _CHUNK + g * _L, _L)] + c0)

    for tr in range(_ID_TR):
      pl.loop(tr * _NCHUNK * 8, (tr + 1) * _NCHUNK * 8)(ioff_body)
      pltpu.async_copy(
          id_flat_hbm.at[ioff_v.at[pl.ds(tr * trw, trw)]],
          ibuf_v.at[pl.ds(tr * trw, trw)], sem_i)

    for q in range(_NCHUNK):
      for g in range(_CHUNK // _L):
        sl = pl.ds(g * _L, _L)
        fsl = pl.ds(q * _CHUNK + g * _L, _L)
        f_v = eidx_v[q, sl] * _ID_NUM + iidx_v[q, sl]
        ebase_v[fsl] = f_v + (f_v >> 7) * 896

    def eoff_body(d):
      tr = d // (_NCHUNK * 8)
      q = (d // 8) % _NCHUNK
      r = d % 8
      c0 = tr * _EXP_TILE_ROW + r * 128
      for g in range(_CHUNK // _L):
        eoff_v[pl.ds(d * _CHUNK + g * _L, _L)] = (
            ebase_v[pl.ds(q * _CHUNK + g * _L, _L)] + c0)

    for tr in range(_EXP_TR):
      pl.loop(tr * _NCHUNK * 8, (tr + 1) * _NCHUNK * 8)(eoff_body)
      pltpu.async_copy(
          exp_flat_hbm.at[eoff_v.at[pl.ds(tr * trw, trw)]],
          ebuf_v.at[pl.ds(tr * trw, trw)], sem_e)

    pltpu.make_async_copy(
        exp_out_hbm.at[pl.ds(0, _EB_WORDS)], ebuf_v, sem_e).wait()
    pltpu.make_async_copy(
        id_out_hbm.at[pl.ds(0, _IB_WORDS)], ibuf_v, sem_i).wait()

    run = _NCHUNK * _CHUNK * 8
    for tr in range(_EXP_TR):
      pltpu.sync_copy(
          ebuf_v.at[pl.ds(tr * run, run)],
          exp_out_hbm.at[pl.ds(tr * _OUT_TILE_ROW + wid * run, run)])
    for tr in range(_ID_TR):
      pltpu.sync_copy(
          ibuf_v.at[pl.ds(tr * run, run)],
          id_out_hbm.at[pl.ds(tr * _OUT_TILE_ROW + wid * run, run)])

  return sc_kernel


def kernel(exp_infor, id_infor, id_table, exp_table):
  e3 = exp_infor.astype(jnp.int32).reshape(_NW, _NCHUNK, _CHUNK)
  i3 = id_infor.astype(jnp.int32).reshape(_NW, _NCHUNK, _CHUNK)

  exp_flat = (exp_table.reshape(6250, 128, _EXP_TR, 8)
              .transpose(2, 0, 3, 1).reshape(-1))
  id_pad = jnp.pad(id_table, ((0, _ID_PAD - _ID_NUM), (0, 0)))
  id_flat = (id_pad.reshape(_ID_PAD // 128, 128, _ID_TR, 8)
             .transpose(2, 0, 3, 1).reshape(-1))

  eo_flat, io_flat = _make_sc_call()(e3, i3, exp_flat, id_flat)

  exp_code = (eo_flat.reshape(_EXP_TR, 128, 8, 128)
              .transpose(1, 3, 0, 2).reshape(_BATCH, _EXP_DIM))
  id_code = (io_flat.reshape(_ID_TR, 128, 8, 128)
             .transpose(1, 3, 0, 2).reshape(_BATCH, _ID_DIM))
  return (exp_code, id_code)
